# baseline (speedup 1.0000x reference)
import functools

import jax
import jax.numpy as jnp
from jax import lax
from jax.experimental import pallas as pl
from jax.experimental.pallas import tpu as pltpu

_LOG2E = 1.4426950408889634
_LN2 = 0.6931471805599453
_BLOCK_N = 33408


def _body(n_total, inv_denom, pred_ref, tgt_ref, out_ref, acc_ref):
    i = pl.program_id(0)
    nblk = pl.num_programs(0)

    @pl.when(i == 0)
    def _init():
        acc_ref[...] = jnp.zeros_like(acc_ref)

    x = pred_ref[...]
    t = tgt_ref[...]
    kk, b = x.shape
    col = i * b + lax.broadcasted_iota(jnp.int32, (1, b), 1)
    valid = col < n_total

    u = jnp.abs(x)
    e = jnp.exp2(-_LOG2E * u)
    lg = jnp.log2(1.0 + e)
    rows = lax.broadcasted_iota(jnp.int32, (kk, b), 0)
    g = jnp.where(rows == t, x, 0.0)
    ones_w = jnp.full((1, kk), 1.0, dtype=jnp.bfloat16)
    row_m = lax.dot(ones_w, (x + u).astype(jnp.bfloat16),
                    preferred_element_type=jnp.float32)
    row_l = lax.dot(ones_w, lg.astype(jnp.bfloat16),
                    preferred_element_type=jnp.float32)
    row_g = lax.dot(ones_w, g.astype(jnp.bfloat16),
                    preferred_element_type=jnp.float32)
    row = 0.5 * row_m + _LN2 * row_l - row_g
    acc_ref[...] += jnp.where(valid, row, 0.0)

    @pl.when(i == nblk - 1)
    def _fin():
        out_ref[0] = jnp.sum(acc_ref[...]) * inv_denom


def kernel(pred, target):
    k, n = pred.shape
    t2 = target.astype(jnp.int32).reshape(1, n)
    grid = pl.cdiv(n, _BLOCK_N)
    out = pl.pallas_call(
        functools.partial(_body, n, 1.0 / (k * n)),
        grid=(grid,),
        in_specs=[
            pl.BlockSpec((k, _BLOCK_N), lambda i: (0, i)),
            pl.BlockSpec((1, _BLOCK_N), lambda i: (0, i)),
        ],
        out_specs=pl.BlockSpec(memory_space=pltpu.SMEM),
        out_shape=jax.ShapeDtypeStruct((1,), jnp.float32),
        scratch_shapes=[pltpu.VMEM((1, _BLOCK_N), jnp.float32)],
    )(pred, t2)
    return out[0]


# speedup vs baseline: 1.0471x; 1.0471x over previous
import functools

import jax
import jax.numpy as jnp
from jax import lax
from jax.experimental import pallas as pl
from jax.experimental.pallas import tpu as pltpu

_LOG2E = 1.4426950408889634
_LN2 = 0.6931471805599453
_BLOCK_N = 20096


def _body(n_total, inv_denom, pred_ref, tgt_ref, out_ref, acc_ref):
    i = pl.program_id(0)
    nblk = pl.num_programs(0)

    @pl.when(i == 0)
    def _init():
        acc_ref[...] = jnp.zeros_like(acc_ref)

    x = pred_ref[...]
    t = tgt_ref[...]
    kk, b = x.shape
    col = i * b + lax.broadcasted_iota(jnp.int32, (1, b), 1)
    valid = col < n_total

    u = jnp.abs(x)
    e = jnp.exp2(-_LOG2E * u)
    lg = jnp.log2(1.0 + e)
    rows = lax.broadcasted_iota(jnp.int32, (kk, b), 0)
    g = jnp.where(rows == t, x, 0.0)
    ones_w = jnp.full((1, kk), 1.0, dtype=jnp.bfloat16)
    row_m = lax.dot(ones_w, (x + u).astype(jnp.bfloat16),
                    preferred_element_type=jnp.float32)
    row_l = lax.dot(ones_w, lg.astype(jnp.bfloat16),
                    preferred_element_type=jnp.float32)
    row_g = lax.dot(ones_w, g.astype(jnp.bfloat16),
                    preferred_element_type=jnp.float32)
    row = 0.5 * row_m + _LN2 * row_l - row_g
    acc_ref[...] += jnp.where(valid, row, 0.0)

    @pl.when(i == nblk - 1)
    def _fin():
        out_ref[0] = jnp.sum(acc_ref[...]) * inv_denom


def kernel(pred, target):
    k, n = pred.shape
    t2 = target.astype(jnp.int32).reshape(1, n)
    grid = pl.cdiv(n, _BLOCK_N)
    out = pl.pallas_call(
        functools.partial(_body, n, 1.0 / (k * n)),
        grid=(grid,),
        in_specs=[
            pl.BlockSpec((k, _BLOCK_N), lambda i: (0, i)),
            pl.BlockSpec((1, _BLOCK_N), lambda i: (0, i)),
        ],
        out_specs=pl.BlockSpec(memory_space=pltpu.SMEM),
        out_shape=jax.ShapeDtypeStruct((1,), jnp.float32),
        scratch_shapes=[pltpu.VMEM((1, _BLOCK_N), jnp.float32)],
    )(pred, t2)
    return out[0]


# speedup vs baseline: 1.1983x; 1.1444x over previous
import functools

import jax
import jax.numpy as jnp
from jax import lax
from jax.experimental import pallas as pl
from jax.experimental.pallas import tpu as pltpu

_LOG2E = 1.4426950408889634
_LN2 = 0.6931471805599453
_BLOCK_N = 25600


def _body(n_total, inv_denom, pred_ref, tgt_ref, out_ref, acc_ref):
    i = pl.program_id(0)
    nblk = pl.num_programs(0)

    @pl.when(i == 0)
    def _init():
        acc_ref[...] = jnp.zeros_like(acc_ref)

    x = pred_ref[...]
    t = tgt_ref[...].reshape(1, -1)
    kk, b = x.shape
    col = i * b + lax.broadcasted_iota(jnp.int32, (1, b), 1)
    valid = col < n_total

    u = jnp.abs(x)
    e = jnp.exp2(-_LOG2E * u)
    lg = jnp.log2(1.0 + e)
    rows = lax.broadcasted_iota(jnp.int32, (kk, b), 0)
    g = jnp.where(rows == t, x, 0.0)
    ones_w = jnp.full((1, kk), 1.0, dtype=jnp.bfloat16)
    row_m = lax.dot(ones_w, (x + u).astype(jnp.bfloat16),
                    preferred_element_type=jnp.float32)
    row_l = lax.dot(ones_w, lg.astype(jnp.bfloat16),
                    preferred_element_type=jnp.float32)
    row_g = lax.dot(ones_w, g.astype(jnp.bfloat16),
                    preferred_element_type=jnp.float32)
    row = 0.5 * row_m + _LN2 * row_l - row_g
    acc_ref[...] += jnp.where(valid, row, 0.0)

    @pl.when(i == nblk - 1)
    def _fin():
        out_ref[0] = jnp.sum(acc_ref[...]) * inv_denom


def kernel(pred, target):
    k, n = pred.shape
    t2 = target.astype(jnp.int32)
    grid = pl.cdiv(n, _BLOCK_N)
    out = pl.pallas_call(
        functools.partial(_body, n, 1.0 / (k * n)),
        grid=(grid,),
        in_specs=[
            pl.BlockSpec((k, _BLOCK_N), lambda i: (0, i)),
            pl.BlockSpec((_BLOCK_N,), lambda i: (i,)),
        ],
        out_specs=pl.BlockSpec(memory_space=pltpu.SMEM),
        out_shape=jax.ShapeDtypeStruct((1,), jnp.float32),
        scratch_shapes=[pltpu.VMEM((1, _BLOCK_N), jnp.float32)],
    )(pred, t2)
    return out[0]


# speedup vs baseline: 1.2153x; 1.0142x over previous
import functools

import jax
import jax.numpy as jnp
from jax import lax
from jax.experimental import pallas as pl
from jax.experimental.pallas import tpu as pltpu

_LOG2E = 1.4426950408889634
_LN2 = 0.6931471805599453
_BLOCK_N = 20480


def _body(n_total, inv_denom, pred_ref, tgt_ref, out_ref, acc_ref):
    i = pl.program_id(0)
    nblk = pl.num_programs(0)

    @pl.when(i == 0)
    def _init():
        acc_ref[...] = jnp.zeros_like(acc_ref)

    x = pred_ref[...]
    t = tgt_ref[...].reshape(1, -1)
    kk, b = x.shape
    col = i * b + lax.broadcasted_iota(jnp.int32, (1, b), 1)
    valid = col < n_total

    u = jnp.abs(x)
    e = jnp.exp2(-_LOG2E * u)
    lg = jnp.log2(1.0 + e)
    rows = lax.broadcasted_iota(jnp.int32, (kk, b), 0)
    g = jnp.where(rows == t, x, 0.0)
    ones_w = jnp.full((1, kk), 1.0, dtype=jnp.bfloat16)
    row_m = lax.dot(ones_w, (x + u).astype(jnp.bfloat16),
                    preferred_element_type=jnp.float32)
    row_l = lax.dot(ones_w, lg.astype(jnp.bfloat16),
                    preferred_element_type=jnp.float32)
    row_g = lax.dot(ones_w, g.astype(jnp.bfloat16),
                    preferred_element_type=jnp.float32)
    row = 0.5 * row_m + _LN2 * row_l - row_g
    acc_ref[...] += jnp.where(valid, row, 0.0)

    @pl.when(i == nblk - 1)
    def _fin():
        out_ref[0] = jnp.sum(acc_ref[...]) * inv_denom


def kernel(pred, target):
    k, n = pred.shape
    t2 = target.astype(jnp.int32)
    grid = pl.cdiv(n, _BLOCK_N)
    out = pl.pallas_call(
        functools.partial(_body, n, 1.0 / (k * n)),
        grid=(grid,),
        in_specs=[
            pl.BlockSpec((k, _BLOCK_N), lambda i: (0, i)),
            pl.BlockSpec((_BLOCK_N,), lambda i: (i,)),
        ],
        out_specs=pl.BlockSpec(memory_space=pltpu.SMEM),
        out_shape=jax.ShapeDtypeStruct((1,), jnp.float32),
        scratch_shapes=[pltpu.VMEM((1, _BLOCK_N), jnp.float32)],
    )(pred, t2)
    return out[0]
